# baseline (speedup 1.0000x reference)
import jax
import jax.numpy as jnp
from jax.experimental import pallas as pl
from jax.experimental.pallas import tpu as pltpu

_CH = 1024
_NBUF = 6
_PREFETCH = 4


def _make_body(batch, seq, d):
    n_w = seq // _CH
    n_t = batch * seq // _CH
    t_per_w = n_t // n_w

    def body(w_hbm, te_hbm, pos_hbm, teo_hbm, wbuf, tbuf, rsw, wsw, rst, wst):
        def w_read(j, slot):
            return pltpu.make_async_copy(
                w_hbm.at[pl.ds(j * _CH, _CH)], wbuf.at[slot], rsw)

        def pos_write(j, b, slot):
            return pltpu.make_async_copy(
                wbuf.at[slot], pos_hbm.at[pl.ds(b * seq + j * _CH, _CH)], wsw)

        def t_read(i, slot):
            return pltpu.make_async_copy(
                te_hbm.at[pl.ds(i * _CH, _CH)], tbuf.at[slot], rst)

        def t_write(i, slot):
            return pltpu.make_async_copy(
                tbuf.at[slot], teo_hbm.at[pl.ds(i * _CH, _CH)], wst)

        tw_pending = []
        pw_pending = []

        w_read(0, 0).start()
        for i in range(min(_PREFETCH, n_t)):
            t_read(i, i % _NBUF).start()

        for i in range(n_t):
            if i % t_per_w == 0:
                j = i // t_per_w
                w_read(j, j % 2).wait()
                if j + 1 < n_w:
                    while pw_pending and pw_pending[0][0] == j - 1:
                        pj, pb = pw_pending.pop(0)
                        pos_write(pj, pb, pj % 2).wait()
                    w_read(j + 1, (j + 1) % 2).start()
                for b in range(batch):
                    pos_write(j, b, j % 2).start()
                    pw_pending.append((j, b))
            t_read(i, i % _NBUF).wait()
            t_write(i, i % _NBUF).start()
            tw_pending.append(i)
            nxt = i + _PREFETCH
            if nxt < n_t:
                if nxt - _NBUF >= 0:
                    oldest = tw_pending.pop(0)
                    t_write(oldest, oldest % _NBUF).wait()
                t_read(nxt, nxt % _NBUF).start()

        for i in tw_pending:
            t_write(i, i % _NBUF).wait()
        for pj, pb in pw_pending:
            pos_write(pj, pb, pj % 2).wait()

    return body


def kernel(tokens, token_embed, W_pos):
    batch, seq, d = token_embed.shape
    w_sliced = W_pos[:seq]
    te_flat = token_embed.reshape(batch * seq, d)
    pos_flat, teo_flat = pl.pallas_call(
        _make_body(batch, seq, d),
        in_specs=[
            pl.BlockSpec(memory_space=pl.ANY),
            pl.BlockSpec(memory_space=pl.ANY),
        ],
        out_specs=[
            pl.BlockSpec(memory_space=pl.ANY),
            pl.BlockSpec(memory_space=pl.ANY),
        ],
        out_shape=[
            jax.ShapeDtypeStruct((batch * seq, d), W_pos.dtype),
            jax.ShapeDtypeStruct((batch * seq, d), token_embed.dtype),
        ],
        scratch_shapes=[
            pltpu.VMEM((2, _CH, 1024), jnp.float32),
            pltpu.VMEM((_NBUF, _CH, 1024), jnp.float32),
            pltpu.SemaphoreType.DMA,
            pltpu.SemaphoreType.DMA,
            pltpu.SemaphoreType.DMA,
            pltpu.SemaphoreType.DMA,
        ],
    )(w_sliced, te_flat)
    return (pos_flat.reshape(batch, seq, d), teo_flat.reshape(batch, seq, d))


# speedup vs baseline: 1.1782x; 1.1782x over previous
import jax
import jax.numpy as jnp
from jax.experimental import pallas as pl


def _fused_kernel(w_ref, te_ref, pos_ref, te_out_ref):
    pos_ref[...] = jnp.broadcast_to(w_ref[...][None, :, :], pos_ref.shape)
    te_out_ref[...] = te_ref[...]


def kernel(tokens, token_embed, W_pos):
    batch, seq, d = token_embed.shape
    block_s = 512
    pos_embed, te_out = pl.pallas_call(
        _fused_kernel,
        grid=(seq // block_s,),
        in_specs=[
            pl.BlockSpec((block_s, d), lambda j: (j, 0)),
            pl.BlockSpec((batch, block_s, d), lambda j: (0, j, 0)),
        ],
        out_specs=[
            pl.BlockSpec((batch, block_s, d), lambda j: (0, j, 0)),
            pl.BlockSpec((batch, block_s, d), lambda j: (0, j, 0)),
        ],
        out_shape=[
            jax.ShapeDtypeStruct((batch, seq, d), W_pos.dtype),
            jax.ShapeDtypeStruct((batch, seq, d), token_embed.dtype),
        ],
    )(W_pos, token_embed)
    return (pos_embed, te_out)


# speedup vs baseline: 1.1789x; 1.0006x over previous
import jax
import jax.numpy as jnp
from jax.experimental import pallas as pl


def _fused_kernel(w_ref, te_ref, pos_ref, te_out_ref):
    pos_ref[...] = jnp.broadcast_to(w_ref[...][None, :, :], pos_ref.shape)
    te_out_ref[...] = te_ref[...]


def kernel(tokens, token_embed, W_pos):
    batch, seq, d = token_embed.shape
    block_s = 512
    n_blocks = seq // block_s
    te_rows = batch * seq // n_blocks
    te_flat = token_embed.reshape(batch * seq, d)
    pos_embed, teo_flat = pl.pallas_call(
        _fused_kernel,
        grid=(n_blocks,),
        in_specs=[
            pl.BlockSpec((block_s, d), lambda j: (j, 0)),
            pl.BlockSpec((te_rows, d), lambda j: (j, 0)),
        ],
        out_specs=[
            pl.BlockSpec((batch, block_s, d), lambda j: (0, j, 0)),
            pl.BlockSpec((te_rows, d), lambda j: (j, 0)),
        ],
        out_shape=[
            jax.ShapeDtypeStruct((batch, seq, d), W_pos.dtype),
            jax.ShapeDtypeStruct((batch * seq, d), token_embed.dtype),
        ],
    )(W_pos, te_flat)
    return (pos_embed, teo_flat.reshape(batch, seq, d))
